# Initial kernel scaffold; baseline (speedup 1.0000x reference)
import jax
import jax.numpy as jnp
from jax.experimental import pallas as pl


def kernel(x, edge_index, batch, W1l, b1l, W1r, W2l, b2l, W2r, Wlin1, blin1, Wlin2, blin2):
    raise NotImplementedError("write your pallas kernel here")


# speedup vs baseline: 6.7750x; 6.7750x over previous
import jax
import jax.numpy as jnp
from jax import lax
from jax.experimental import pallas as pl
from jax.experimental.pallas import tpu as pltpu
from jax.experimental.pallas import tpu_sc as plsc

N = 10000
E = 320000
D = 128
G = 64
C = 16

NC = 2
NS = 16
NW = NC * NS
EW = E // NW
CH = 80
NCH = EW // CH
RCH = 200
NRC = N // RCH

R = 400
NB = N // R


def _tile_chunks(sid, fn):
    for k in range(NRC // NS + 1):
        ch = k * NS + sid
        if k * NS + NS <= NRC:
            fn(ch)
        else:
            @pl.when(ch < NRC)
            def _():
                fn(ch)


def _make_sc_agg():
    out_type = jax.ShapeDtypeStruct((NC, N, D), jnp.float32)
    scratch = [
        pltpu.VMEM_SHARED((N, D), jnp.float32),
        pltpu.VMEM((CH,), jnp.int32),
        pltpu.VMEM((CH,), jnp.int32),
        pltpu.VMEM((CH,), jnp.int32),
        pltpu.VMEM((CH, D), jnp.float32),
        pltpu.VMEM((CH, D), jnp.float32),
        pltpu.VMEM((RCH, D), jnp.float32),
        pltpu.SemaphoreType.DMA,
        pltpu.SemaphoreType.DMA,
    ]
    mesh = plsc.VectorSubcoreMesh(core_axis_name="c", subcore_axis_name="s",
                                  num_cores=NC, num_subcores=NS)

    def body(table, src_hbm, dst_hbm, out_acc,
             acc_s, ib0, ib1, db, rb0, rb1, zb, sem0, sem1):
        cid = lax.axis_index("c")
        sid = lax.axis_index("s")
        wid = cid * NS + sid

        def zrow(r, _):
            for j in range(D // 16):
                zb[r, pl.ds(j * 16, 16)] = jnp.zeros((16,), jnp.float32)
            return 0
        lax.fori_loop(0, RCH, zrow, 0)
        _tile_chunks(sid, lambda ch: pltpu.sync_copy(
            zb, acc_s.at[pl.ds(ch * RCH, RCH)]))
        plsc.subcore_barrier()

        e0 = wid * EW

        def scat(rb, c):
            pltpu.sync_copy(dst_hbm.at[pl.ds(e0 + c * CH, CH)], db)
            pltpu.sync_copy(rb, acc_s.at[db], add=True)

        def fetch(ib, rb, sem, c):
            pltpu.sync_copy(src_hbm.at[pl.ds(e0 + c * CH, CH)], ib)
            pltpu.async_copy(table.at[ib], rb, sem)

        fetch(ib0, rb0, sem0, 0)

        def step(c, _):
            fetch(ib1, rb1, sem1, c + 1)
            pltpu.make_async_copy(table.at[ib0], rb0, sem0).wait()
            scat(rb0, c)
            fetch(ib0, rb0, sem0, c + 2)
            pltpu.make_async_copy(table.at[ib1], rb1, sem1).wait()
            scat(rb1, c + 1)
            return 0
        lax.fori_loop(0, (NCH - 1) // 2, lambda p, _: step(2 * p, _), 0)
        pltpu.make_async_copy(table.at[ib0], rb0, sem0).wait()
        scat(rb0, NCH - 1)

        plsc.subcore_barrier()

        def rd_acc(ch):
            pltpu.sync_copy(acc_s.at[pl.ds(ch * RCH, RCH)], zb)
            pltpu.sync_copy(zb, out_acc.at[cid, pl.ds(ch * RCH, RCH)])
        _tile_chunks(sid, rd_acc)

    return pl.kernel(body, out_type=out_type, mesh=mesh, scratch_types=scratch)


def _make_sc_count():
    out_type = jax.ShapeDtypeStruct((NC, N, D), jnp.float32)
    scratch = [
        pltpu.VMEM_SHARED((N, D), jnp.float32),
        pltpu.VMEM((CH,), jnp.int32),
        pltpu.VMEM((CH, D), jnp.float32),
        pltpu.VMEM((RCH, D), jnp.float32),
    ]
    mesh = plsc.VectorSubcoreMesh(core_axis_name="c", subcore_axis_name="s",
                                  num_cores=NC, num_subcores=NS)

    def body(dst_hbm, out_cnt, cnt_s, db, ones_v, cb):
        cid = lax.axis_index("c")
        sid = lax.axis_index("s")
        wid = cid * NS + sid

        def zrow(r, _):
            for j in range(D // 16):
                cb[r, pl.ds(j * 16, 16)] = jnp.zeros((16,), jnp.float32)
            return 0
        lax.fori_loop(0, RCH, zrow, 0)
        _tile_chunks(sid, lambda ch: pltpu.sync_copy(
            cb, cnt_s.at[pl.ds(ch * RCH, RCH)]))

        def orow(r, _):
            for j in range(D // 16):
                ones_v[r, pl.ds(j * 16, 16)] = jnp.ones((16,), jnp.float32)
            return 0
        lax.fori_loop(0, CH, orow, 0)
        plsc.subcore_barrier()

        e0 = wid * EW

        def step(c, _):
            pltpu.sync_copy(dst_hbm.at[pl.ds(e0 + c * CH, CH)], db)
            pltpu.sync_copy(ones_v, cnt_s.at[db], add=True)
            return 0
        lax.fori_loop(0, NCH, step, 0)

        plsc.subcore_barrier()

        def rd_cnt(ch):
            pltpu.sync_copy(cnt_s.at[pl.ds(ch * RCH, RCH)], cb)
            pltpu.sync_copy(cb, out_cnt.at[cid, pl.ds(ch * RCH, RCH)])
        _tile_chunks(sid, rd_cnt)

    return pl.kernel(body, out_type=out_type, mesh=mesh, scratch_types=scratch)


_sc_agg = _make_sc_agg()
_sc_count = _make_sc_count()


def _tc_layer1(a0, a1, ca, cb, x, wl, wr, b):
    def body(a0r, a1r, car, cbr, xr, wlr, wrr, br, o):
        cnt = car[...][:, :1] + cbr[...][:, :1]
        inv = 1.0 / jnp.maximum(cnt, 1.0)
        agg = (a0r[...] + a1r[...]) * inv
        h = (jnp.dot(agg, wlr[...], preferred_element_type=jnp.float32)
             + br[...]
             + jnp.dot(xr[...], wrr[...], preferred_element_type=jnp.float32))
        o[...] = jnp.maximum(h, 0.0)

    row = pl.BlockSpec((R, D), lambda i: (i, 0))
    cntspec = pl.BlockSpec((R, 16), lambda i: (i, 0))
    full = pl.BlockSpec((D, D), lambda i: (0, 0))
    bias = pl.BlockSpec((1, D), lambda i: (0, 0))
    return pl.pallas_call(
        body,
        grid=(NB,),
        in_specs=[row, row, cntspec, cntspec, row, full, full, bias],
        out_specs=row,
        out_shape=jax.ShapeDtypeStruct((N, D), jnp.float32),
    )(a0, a1, ca, cb, x, wl, wr, b)


def _tc_layer2(b0, b1, ca, cb, h1, bt, wl, wr, b2, wlin1, blin1, wlin2, blin2):
    def body(b0r, b1r, car, cbr, h1r, btr, wlr, wrr, b2r,
             w1r_, b1r_, w2r_, b2r_, o, gsum, gcnt):
        i = pl.program_id(0)

        @pl.when(i == 0)
        def _init():
            gsum[...] = jnp.zeros((G, D), jnp.float32)
            gcnt[...] = jnp.zeros((G, D), jnp.float32)

        cnt = car[...][:, :1] + cbr[...][:, :1]
        inv = 1.0 / jnp.maximum(cnt, 1.0)
        agg = (b0r[...] + b1r[...]) * inv
        h2 = jnp.maximum(
            jnp.dot(agg, wlr[...], preferred_element_type=jnp.float32)
            + b2r[...]
            + jnp.dot(h1r[...], wrr[...], preferred_element_type=jnp.float32),
            0.0)

        ids = lax.broadcasted_iota(jnp.int32, (G, R), 0)
        mask = (ids == btr[0]).astype(jnp.float32)
        gsum[...] += jnp.dot(mask, h2, preferred_element_type=jnp.float32)
        gcnt[...] += jnp.broadcast_to(
            jnp.sum(mask, axis=1, keepdims=True), (G, D))

        @pl.when(i == NB - 1)
        def _fin():
            g = gsum[...] / jnp.maximum(gcnt[...], 1.0)
            t = (jnp.dot(g, w1r_[...], preferred_element_type=jnp.float32)
                 + b1r_[...])
            o[...] = (jnp.dot(t, w2r_[...], preferred_element_type=jnp.float32)
                      + b2r_[...])

    row = pl.BlockSpec((R, D), lambda i: (i, 0))
    cntspec = pl.BlockSpec((R, 16), lambda i: (i, 0))
    full = pl.BlockSpec((D, D), lambda i: (0, 0))
    fix = lambda s: pl.BlockSpec(s, lambda i: tuple(0 for _ in s))
    return pl.pallas_call(
        body,
        grid=(NB,),
        in_specs=[row, row, cntspec, cntspec, row,
                  pl.BlockSpec((1, 1, R), lambda i: (i, 0, 0)),
                  full, full, fix((1, D)),
                  fix((D, G)), fix((1, G)), fix((G, C)), fix((1, C))],
        out_specs=fix((G, C)),
        out_shape=jax.ShapeDtypeStruct((G, C), jnp.float32),
        scratch_shapes=[pltpu.VMEM((G, D), jnp.float32),
                        pltpu.VMEM((G, D), jnp.float32)],
    )(b0, b1, ca, cb, h1, bt, wl, wr, b2, wlin1, blin1, wlin2, blin2)


def kernel(x, edge_index, batch, W1l, b1l, W1r, W2l, b2l, W2r,
           Wlin1, blin1, Wlin2, blin2):
    src = edge_index[0]
    dst = edge_index[1]

    cnt2 = _sc_count(dst)
    accA = _sc_agg(x, src, dst)
    ca, cb = cnt2[0, :, :16], cnt2[1, :, :16]
    h1 = _tc_layer1(accA[0], accA[1], ca, cb, x,
                    W1l.T, W1r.T, b1l.reshape(1, D))
    accB = _sc_agg(h1, src, dst)
    bt = batch.reshape(NB, 1, R)
    out = _tc_layer2(accB[0], accB[1], ca, cb, h1, bt,
                     W2l.T, W2r.T, b2l.reshape(1, D),
                     Wlin1.T, blin1.reshape(1, G), Wlin2.T, blin2.reshape(1, C))
    return out


# speedup vs baseline: 7.8827x; 1.1635x over previous
import jax
import jax.numpy as jnp
from jax import lax
from jax.experimental import pallas as pl
from jax.experimental.pallas import tpu as pltpu
from jax.experimental.pallas import tpu_sc as plsc

N = 10000
E = 320000
D = 128
G = 64
C = 16

NC = 2
NS = 16
NW = NC * NS
EW = E // NW
CH = 80
NCH = EW // CH
RCH = 200
NRC = N // RCH

R = 400
NB = N // R


def _tile_chunks(sid, fn):
    for k in range(NRC // NS + 1):
        ch = k * NS + sid
        if k * NS + NS <= NRC:
            fn(ch)
        else:
            @pl.when(ch < NRC)
            def _():
                fn(ch)


def _make_sc_agg():
    out_type = jax.ShapeDtypeStruct((NC, N, D), jnp.float32)
    scratch = [
        pltpu.VMEM_SHARED((N, D), jnp.float32),
        pltpu.VMEM((CH,), jnp.int32),
        pltpu.VMEM((CH,), jnp.int32),
        pltpu.VMEM((CH,), jnp.int32),
        pltpu.VMEM((CH, D), jnp.float32),
        pltpu.VMEM((CH, D), jnp.float32),
        pltpu.VMEM((RCH, D), jnp.float32),
        pltpu.SemaphoreType.DMA,
        pltpu.SemaphoreType.DMA,
    ]
    mesh = plsc.VectorSubcoreMesh(core_axis_name="c", subcore_axis_name="s",
                                  num_cores=NC, num_subcores=NS)

    def body(table, src_hbm, dst_hbm, out_acc,
             acc_s, ib0, ib1, db, rb0, rb1, zb, sem0, sem1):
        cid = lax.axis_index("c")
        sid = lax.axis_index("s")
        wid = cid * NS + sid

        def zrow(r, _):
            for j in range(D // 16):
                zb[r, pl.ds(j * 16, 16)] = jnp.zeros((16,), jnp.float32)
            return 0
        lax.fori_loop(0, RCH, zrow, 0)
        _tile_chunks(sid, lambda ch: pltpu.sync_copy(
            zb, acc_s.at[pl.ds(ch * RCH, RCH)]))
        plsc.subcore_barrier()

        e0 = wid * EW

        def scat(rb, c):
            pltpu.sync_copy(dst_hbm.at[pl.ds(e0 + c * CH, CH)], db)
            pltpu.sync_copy(rb, acc_s.at[db], add=True)

        def fetch(ib, rb, sem, c):
            pltpu.sync_copy(src_hbm.at[pl.ds(e0 + c * CH, CH)], ib)
            pltpu.async_copy(table.at[ib], rb, sem)

        fetch(ib0, rb0, sem0, 0)

        def step(c, _):
            fetch(ib1, rb1, sem1, c + 1)
            pltpu.make_async_copy(table.at[ib0], rb0, sem0).wait()
            scat(rb0, c)
            fetch(ib0, rb0, sem0, c + 2)
            pltpu.make_async_copy(table.at[ib1], rb1, sem1).wait()
            scat(rb1, c + 1)
            return 0
        lax.fori_loop(0, (NCH - 1) // 2, lambda p, _: step(2 * p, _), 0)
        pltpu.make_async_copy(table.at[ib0], rb0, sem0).wait()
        scat(rb0, NCH - 1)

        plsc.subcore_barrier()

        def rd_acc(ch):
            pltpu.sync_copy(acc_s.at[pl.ds(ch * RCH, RCH)], zb)
            pltpu.sync_copy(zb, out_acc.at[cid, pl.ds(ch * RCH, RCH)])
        _tile_chunks(sid, rd_acc)

    return pl.kernel(body, out_type=out_type, mesh=mesh, scratch_types=scratch)


def _make_sc_count():
    out_type = jax.ShapeDtypeStruct((NC * N,), jnp.float32)
    scratch = [
        pltpu.VMEM_SHARED((N,), jnp.float32),
        pltpu.VMEM((CH,), jnp.int32),
        pltpu.VMEM((CH,), jnp.int32),
        pltpu.VMEM((CH,), jnp.float32),
        pltpu.VMEM((RCH,), jnp.float32),
        pltpu.SemaphoreType.DMA,
        pltpu.SemaphoreType.DMA,
    ]
    mesh = plsc.VectorSubcoreMesh(core_axis_name="c", subcore_axis_name="s",
                                  num_cores=NC, num_subcores=NS)

    def body(dst_hbm, ones_hbm, zeros_hbm, out_cnt,
             cnt_s, db0, db1, ones_v, cb, semd0, semd1):
        cid = lax.axis_index("c")
        sid = lax.axis_index("s")
        wid = cid * NS + sid

        pltpu.sync_copy(zeros_hbm, cb)
        _tile_chunks(sid, lambda ch: pltpu.sync_copy(
            cb, cnt_s.at[pl.ds(ch * RCH, RCH)]))
        pltpu.sync_copy(ones_hbm, ones_v)
        plsc.subcore_barrier()

        e0 = wid * EW

        def ld(db, sem, c):
            pltpu.async_copy(dst_hbm.at[pl.ds(e0 + c * CH, CH)], db, sem)

        def wt(db, sem):
            pltpu.make_async_copy(dst_hbm.at[pl.ds(e0, CH)], db, sem).wait()

        ld(db0, semd0, 0)
        ld(db1, semd1, 1)

        def step(c, _):
            wt(db0, semd0)
            pltpu.sync_copy(ones_v, cnt_s.at[db0], add=True)
            ld(db0, semd0, c + 2)
            wt(db1, semd1)
            pltpu.sync_copy(ones_v, cnt_s.at[db1], add=True)
            ld(db1, semd1, c + 3)
            return 0
        lax.fori_loop(0, (NCH - 1) // 2, lambda p, _: step(2 * p, _), 0)
        wt(db0, semd0)
        pltpu.sync_copy(ones_v, cnt_s.at[db0], add=True)
        wt(db1, semd1)

        plsc.subcore_barrier()

        def rd_cnt(ch):
            pltpu.sync_copy(cnt_s.at[pl.ds(ch * RCH, RCH)], cb)
            pltpu.sync_copy(cb, out_cnt.at[pl.ds(cid * N + ch * RCH, RCH)])
        _tile_chunks(sid, rd_cnt)

    return pl.kernel(body, out_type=out_type, mesh=mesh, scratch_types=scratch)


_sc_agg = _make_sc_agg()
_sc_count = _make_sc_count()


def _tc_layer1(a0, a1, ca, cb, x, wl, wr, b):
    def body(a0r, a1r, car, cbr, xr, wlr, wrr, br, o):
        cnt = car[...] + cbr[...]
        inv = 1.0 / jnp.maximum(cnt, 1.0)
        agg = (a0r[...] + a1r[...]) * inv
        h = (jnp.dot(agg, wlr[...], preferred_element_type=jnp.float32)
             + br[...]
             + jnp.dot(xr[...], wrr[...], preferred_element_type=jnp.float32))
        o[...] = jnp.maximum(h, 0.0)

    row = pl.BlockSpec((R, D), lambda i: (i, 0))
    cntspec = pl.BlockSpec((R, 1), lambda i: (i, 0))
    full = pl.BlockSpec((D, D), lambda i: (0, 0))
    bias = pl.BlockSpec((1, D), lambda i: (0, 0))
    return pl.pallas_call(
        body,
        grid=(NB,),
        in_specs=[row, row, cntspec, cntspec, row, full, full, bias],
        out_specs=row,
        out_shape=jax.ShapeDtypeStruct((N, D), jnp.float32),
    )(a0, a1, ca, cb, x, wl, wr, b)


def _tc_layer2(b0, b1, ca, cb, h1, bt, wl, wr, b2, wlin1, blin1, wlin2, blin2):
    def body(b0r, b1r, car, cbr, h1r, btr, wlr, wrr, b2r,
             w1r_, b1r_, w2r_, b2r_, o, gsum, gcnt):
        i = pl.program_id(0)

        @pl.when(i == 0)
        def _init():
            gsum[...] = jnp.zeros((G, D), jnp.float32)
            gcnt[...] = jnp.zeros((G, D), jnp.float32)

        cnt = car[...] + cbr[...]
        inv = 1.0 / jnp.maximum(cnt, 1.0)
        agg = (b0r[...] + b1r[...]) * inv
        h2 = jnp.maximum(
            jnp.dot(agg, wlr[...], preferred_element_type=jnp.float32)
            + b2r[...]
            + jnp.dot(h1r[...], wrr[...], preferred_element_type=jnp.float32),
            0.0)

        ids = lax.broadcasted_iota(jnp.int32, (G, R), 0)
        mask = (ids == btr[0]).astype(jnp.float32)
        gsum[...] += jnp.dot(mask, h2, preferred_element_type=jnp.float32)
        gcnt[...] += jnp.broadcast_to(
            jnp.sum(mask, axis=1, keepdims=True), (G, D))

        @pl.when(i == NB - 1)
        def _fin():
            g = gsum[...] / jnp.maximum(gcnt[...], 1.0)
            t = (jnp.dot(g, w1r_[...], preferred_element_type=jnp.float32)
                 + b1r_[...])
            o[...] = (jnp.dot(t, w2r_[...], preferred_element_type=jnp.float32)
                      + b2r_[...])

    row = pl.BlockSpec((R, D), lambda i: (i, 0))
    cntspec = pl.BlockSpec((R, 1), lambda i: (i, 0))
    full = pl.BlockSpec((D, D), lambda i: (0, 0))
    fix = lambda s: pl.BlockSpec(s, lambda i: tuple(0 for _ in s))
    return pl.pallas_call(
        body,
        grid=(NB,),
        in_specs=[row, row, cntspec, cntspec, row,
                  pl.BlockSpec((1, 1, R), lambda i: (i, 0, 0)),
                  full, full, fix((1, D)),
                  fix((D, G)), fix((1, G)), fix((G, C)), fix((1, C))],
        out_specs=fix((G, C)),
        out_shape=jax.ShapeDtypeStruct((G, C), jnp.float32),
        scratch_shapes=[pltpu.VMEM((G, D), jnp.float32),
                        pltpu.VMEM((G, D), jnp.float32)],
    )(b0, b1, ca, cb, h1, bt, wl, wr, b2, wlin1, blin1, wlin2, blin2)


def kernel(x, edge_index, batch, W1l, b1l, W1r, W2l, b2l, W2r,
           Wlin1, blin1, Wlin2, blin2):
    src = edge_index[0]
    dst = edge_index[1]

    dst_p = jnp.concatenate([dst, jnp.zeros((CH,), jnp.int32)])
    cnt2 = _sc_count(dst_p, jnp.ones((CH,), jnp.float32),
                     jnp.zeros((RCH,), jnp.float32))
    accA = _sc_agg(x, src, dst)
    ca = cnt2[0:N].reshape(N, 1)
    cb = cnt2[N:].reshape(N, 1)
    h1 = _tc_layer1(accA[0], accA[1], ca, cb, x,
                    W1l.T, W1r.T, b1l.reshape(1, D))
    accB = _sc_agg(h1, src, dst)
    bt = batch.reshape(NB, 1, R)
    out = _tc_layer2(accB[0], accB[1], ca, cb, h1, bt,
                     W2l.T, W2r.T, b2l.reshape(1, D),
                     Wlin1.T, blin1.reshape(1, G), Wlin2.T, blin2.reshape(1, C))
    return out


# speedup vs baseline: 10.5161x; 1.3341x over previous
import jax
import jax.numpy as jnp
from jax import lax
from jax.experimental import pallas as pl
from jax.experimental.pallas import tpu as pltpu
from jax.experimental.pallas import tpu_sc as plsc

N = 10000
E = 320000
D = 128
G = 64
C = 16

NC = 2
NS = 16
NW = NC * NS
EW = E // NW
CH = 80
NCH = EW // CH
RCH = 200
NRC = N // RCH

R = 400
NB = N // R


def _tile_chunks(sid, fn):
    for k in range(NRC // NS + 1):
        ch = k * NS + sid
        if k * NS + NS <= NRC:
            fn(ch)
        else:
            @pl.when(ch < NRC)
            def _():
                fn(ch)


def _make_sc_agg():
    out_type = jax.ShapeDtypeStruct((NC, N, D), jnp.float32)
    scratch = [
        pltpu.VMEM_SHARED((N, D), jnp.float32),
        pltpu.VMEM((CH,), jnp.int32),
        pltpu.VMEM((CH,), jnp.int32),
        pltpu.VMEM((CH,), jnp.int32),
        pltpu.VMEM((CH,), jnp.int32),
        pltpu.VMEM((CH, D), jnp.float32),
        pltpu.VMEM((CH, D), jnp.float32),
        pltpu.VMEM((RCH, D), jnp.float32),
        pltpu.SemaphoreType.DMA,
        pltpu.SemaphoreType.DMA,
        pltpu.SemaphoreType.DMA,
        pltpu.SemaphoreType.DMA,
        pltpu.SemaphoreType.DMA,
        pltpu.SemaphoreType.DMA,
    ]
    mesh = plsc.VectorSubcoreMesh(core_axis_name="c", subcore_axis_name="s",
                                  num_cores=NC, num_subcores=NS)

    def body(table, src_hbm, dst_hbm, out_acc,
             acc_s, ib0, ib1, db0, db1, rb0, rb1, zb,
             sem0, sem1, semi0, semi1, semd0, semd1):
        cid = lax.axis_index("c")
        sid = lax.axis_index("s")
        wid = cid * NS + sid

        def zrow(r, _):
            for j in range(D // 16):
                zb[r, pl.ds(j * 16, 16)] = jnp.zeros((16,), jnp.float32)
            return 0
        lax.fori_loop(0, RCH, zrow, 0)
        _tile_chunks(sid, lambda ch: pltpu.sync_copy(
            zb, acc_s.at[pl.ds(ch * RCH, RCH)]))
        plsc.subcore_barrier()

        e0 = wid * EW

        def ld(buf, sem, arr, c):
            pltpu.async_copy(arr.at[pl.ds(e0 + c * CH, CH)], buf, sem)

        def wt_idx(buf, sem, arr):
            pltpu.make_async_copy(arr.at[pl.ds(e0, CH)], buf, sem).wait()

        def gather(ib, rb, sem):
            pltpu.async_copy(table.at[ib], rb, sem)

        def wt_rows(ib, rb, sem):
            pltpu.make_async_copy(table.at[ib], rb, sem).wait()

        ld(ib0, semi0, src_hbm, 0)
        ld(db0, semd0, dst_hbm, 0)
        ld(ib1, semi1, src_hbm, 1)
        ld(db1, semd1, dst_hbm, 1)
        wt_idx(ib0, semi0, src_hbm)
        gather(ib0, rb0, sem0)

        def step(c, _):
            wt_idx(ib1, semi1, src_hbm)
            gather(ib1, rb1, sem1)
            wt_rows(ib0, rb0, sem0)
            ld(ib0, semi0, src_hbm, c + 2)
            wt_idx(db0, semd0, dst_hbm)
            pltpu.sync_copy(rb0, acc_s.at[db0], add=True)
            ld(db0, semd0, dst_hbm, c + 2)
            wt_idx(ib0, semi0, src_hbm)
            gather(ib0, rb0, sem0)
            wt_rows(ib1, rb1, sem1)
            ld(ib1, semi1, src_hbm, c + 3)
            wt_idx(db1, semd1, dst_hbm)
            pltpu.sync_copy(rb1, acc_s.at[db1], add=True)
            ld(db1, semd1, dst_hbm, c + 3)
            return 0
        lax.fori_loop(0, (NCH - 1) // 2, lambda p, _: step(2 * p, _), 0)
        wt_rows(ib0, rb0, sem0)
        wt_idx(db0, semd0, dst_hbm)
        pltpu.sync_copy(rb0, acc_s.at[db0], add=True)
        wt_idx(ib1, semi1, src_hbm)
        wt_idx(db1, semd1, dst_hbm)

        plsc.subcore_barrier()

        def rd_acc(ch):
            pltpu.sync_copy(acc_s.at[pl.ds(ch * RCH, RCH)], zb)
            pltpu.sync_copy(zb, out_acc.at[cid, pl.ds(ch * RCH, RCH)])
        _tile_chunks(sid, rd_acc)

    return pl.kernel(body, out_type=out_type, mesh=mesh, scratch_types=scratch)


def _make_sc_count():
    out_type = jax.ShapeDtypeStruct((NC * N,), jnp.float32)
    scratch = [
        pltpu.VMEM_SHARED((N,), jnp.float32),
        pltpu.VMEM((CH,), jnp.int32),
        pltpu.VMEM((CH,), jnp.int32),
        pltpu.VMEM((CH,), jnp.float32),
        pltpu.VMEM((RCH,), jnp.float32),
        pltpu.SemaphoreType.DMA,
        pltpu.SemaphoreType.DMA,
    ]
    mesh = plsc.VectorSubcoreMesh(core_axis_name="c", subcore_axis_name="s",
                                  num_cores=NC, num_subcores=NS)

    def body(dst_hbm, ones_hbm, zeros_hbm, out_cnt,
             cnt_s, db0, db1, ones_v, cb, semd0, semd1):
        cid = lax.axis_index("c")
        sid = lax.axis_index("s")
        wid = cid * NS + sid

        pltpu.sync_copy(zeros_hbm, cb)
        _tile_chunks(sid, lambda ch: pltpu.sync_copy(
            cb, cnt_s.at[pl.ds(ch * RCH, RCH)]))
        pltpu.sync_copy(ones_hbm, ones_v)
        plsc.subcore_barrier()

        e0 = wid * EW

        def ld(db, sem, c):
            pltpu.async_copy(dst_hbm.at[pl.ds(e0 + c * CH, CH)], db, sem)

        def wt(db, sem):
            pltpu.make_async_copy(dst_hbm.at[pl.ds(e0, CH)], db, sem).wait()

        ld(db0, semd0, 0)
        ld(db1, semd1, 1)

        def step(c, _):
            wt(db0, semd0)
            pltpu.sync_copy(ones_v, cnt_s.at[db0], add=True)
            ld(db0, semd0, c + 2)
            wt(db1, semd1)
            pltpu.sync_copy(ones_v, cnt_s.at[db1], add=True)
            ld(db1, semd1, c + 3)
            return 0
        lax.fori_loop(0, (NCH - 1) // 2, lambda p, _: step(2 * p, _), 0)
        wt(db0, semd0)
        pltpu.sync_copy(ones_v, cnt_s.at[db0], add=True)
        wt(db1, semd1)

        plsc.subcore_barrier()

        def rd_cnt(ch):
            pltpu.sync_copy(cnt_s.at[pl.ds(ch * RCH, RCH)], cb)
            pltpu.sync_copy(cb, out_cnt.at[pl.ds(cid * N + ch * RCH, RCH)])
        _tile_chunks(sid, rd_cnt)

    return pl.kernel(body, out_type=out_type, mesh=mesh, scratch_types=scratch)


_sc_agg = _make_sc_agg()
_sc_count = _make_sc_count()


def _tc_layer1(a0, a1, ca, cb, x, wl, wr, b):
    def body(a0r, a1r, car, cbr, xr, wlr, wrr, br, o):
        cnt = car[...] + cbr[...]
        inv = 1.0 / jnp.maximum(cnt, 1.0)
        agg = (a0r[...] + a1r[...]) * inv
        h = (jnp.dot(agg, wlr[...], preferred_element_type=jnp.float32)
             + br[...]
             + jnp.dot(xr[...], wrr[...], preferred_element_type=jnp.float32))
        o[...] = jnp.maximum(h, 0.0)

    row = pl.BlockSpec((R, D), lambda i: (i, 0))
    cntspec = pl.BlockSpec((R, 1), lambda i: (i, 0))
    full = pl.BlockSpec((D, D), lambda i: (0, 0))
    bias = pl.BlockSpec((1, D), lambda i: (0, 0))
    return pl.pallas_call(
        body,
        grid=(NB,),
        in_specs=[row, row, cntspec, cntspec, row, full, full, bias],
        out_specs=row,
        out_shape=jax.ShapeDtypeStruct((N, D), jnp.float32),
    )(a0, a1, ca, cb, x, wl, wr, b)


def _tc_layer2(b0, b1, ca, cb, h1, bt, wl, wr, b2, wlin1, blin1, wlin2, blin2):
    def body(b0r, b1r, car, cbr, h1r, btr, wlr, wrr, b2r,
             w1r_, b1r_, w2r_, b2r_, o, gsum, gcnt):
        i = pl.program_id(0)

        @pl.when(i == 0)
        def _init():
            gsum[...] = jnp.zeros((G, D), jnp.float32)
            gcnt[...] = jnp.zeros((G, D), jnp.float32)

        cnt = car[...] + cbr[...]
        inv = 1.0 / jnp.maximum(cnt, 1.0)
        agg = (b0r[...] + b1r[...]) * inv
        h2 = jnp.maximum(
            jnp.dot(agg, wlr[...], preferred_element_type=jnp.float32)
            + b2r[...]
            + jnp.dot(h1r[...], wrr[...], preferred_element_type=jnp.float32),
            0.0)

        ids = lax.broadcasted_iota(jnp.int32, (G, R), 0)
        mask = (ids == btr[0]).astype(jnp.float32)
        gsum[...] += jnp.dot(mask, h2, preferred_element_type=jnp.float32)
        gcnt[...] += jnp.broadcast_to(
            jnp.sum(mask, axis=1, keepdims=True), (G, D))

        @pl.when(i == NB - 1)
        def _fin():
            g = gsum[...] / jnp.maximum(gcnt[...], 1.0)
            t = (jnp.dot(g, w1r_[...], preferred_element_type=jnp.float32)
                 + b1r_[...])
            o[...] = (jnp.dot(t, w2r_[...], preferred_element_type=jnp.float32)
                      + b2r_[...])

    row = pl.BlockSpec((R, D), lambda i: (i, 0))
    cntspec = pl.BlockSpec((R, 1), lambda i: (i, 0))
    full = pl.BlockSpec((D, D), lambda i: (0, 0))
    fix = lambda s: pl.BlockSpec(s, lambda i: tuple(0 for _ in s))
    return pl.pallas_call(
        body,
        grid=(NB,),
        in_specs=[row, row, cntspec, cntspec, row,
                  pl.BlockSpec((1, 1, R), lambda i: (i, 0, 0)),
                  full, full, fix((1, D)),
                  fix((D, G)), fix((1, G)), fix((G, C)), fix((1, C))],
        out_specs=fix((G, C)),
        out_shape=jax.ShapeDtypeStruct((G, C), jnp.float32),
        scratch_shapes=[pltpu.VMEM((G, D), jnp.float32),
                        pltpu.VMEM((G, D), jnp.float32)],
    )(b0, b1, ca, cb, h1, bt, wl, wr, b2, wlin1, blin1, wlin2, blin2)


def kernel(x, edge_index, batch, W1l, b1l, W1r, W2l, b2l, W2r,
           Wlin1, blin1, Wlin2, blin2):
    src = edge_index[0]
    dst = edge_index[1]

    src_p = jnp.concatenate([src, jnp.zeros((CH,), jnp.int32)])
    dst_p = jnp.concatenate([dst, jnp.zeros((CH,), jnp.int32)])
    cnt2 = _sc_count(dst_p, jnp.ones((CH,), jnp.float32),
                     jnp.zeros((RCH,), jnp.float32))
    accA = _sc_agg(x, src_p, dst_p)
    ca = cnt2[0:N].reshape(N, 1)
    cb = cnt2[N:].reshape(N, 1)
    h1 = _tc_layer1(accA[0], accA[1], ca, cb, x,
                    W1l.T, W1r.T, b1l.reshape(1, D))
    accB = _sc_agg(h1, src_p, dst_p)
    bt = batch.reshape(NB, 1, R)
    out = _tc_layer2(accB[0], accB[1], ca, cb, h1, bt,
                     W2l.T, W2r.T, b2l.reshape(1, D),
                     Wlin1.T, blin1.reshape(1, G), Wlin2.T, blin2.reshape(1, C))
    return out
